# baseline (speedup 1.0000x reference)
import functools

import jax
import jax.numpy as jnp
from jax import lax
from jax.experimental import pallas as pl
from jax.experimental.pallas import tpu as pltpu
from jax.experimental.pallas import tpu_sc as plsc

_NC = 2
_NS = 16
_NW = _NC * _NS

_G = 512
_NI = 8
_NJ = 4
_GB = _G // 128
_P = 129


@functools.partial(jax.jit, static_argnums=(1, 2))
def _sc_transpose(table_t, R, D):
    C = 504
    cpw = (R // (_NW * 8)) * 8
    nch = cpw // C
    rem = R - _NW * cpw
    mesh = plsc.VectorSubcoreMesh(core_axis_name="c", subcore_axis_name="s")

    @functools.partial(
        pl.kernel,
        mesh=mesh,
        out_type=jax.ShapeDtypeStruct((R, D), jnp.float32),
        scratch_types=[
            pltpu.VMEM((D, C), jnp.float32),
            pltpu.VMEM((D, C), jnp.float32),
            pltpu.VMEM((C, D + 1), jnp.float32),
            pltpu.VMEM((C, D + 1), jnp.float32),
            pltpu.SemaphoreType.DMA((2,)),
            pltpu.SemaphoreType.DMA((2,)),
        ],
        compiler_params=pltpu.CompilerParams(
            use_tc_tiling_on_sc=False, needs_layout_passes=False),
    )
    def k(tt_hbm, out_hbm, rb0, rb1, st0, st1, rsem, wsem):
        wid = lax.axis_index("s") * _NC + lax.axis_index("c")
        base = wid * cpw
        rbufs = (rb0, rb1)
        sbufs = (st0, st1)
        lane = jnp.arange(16, dtype=jnp.int32)
        fsp = [jnp.zeros((16,), jnp.int32) + f for f in range(D)]

        def rd_start(c0, b):
            pltpu.make_async_copy(
                tt_hbm.at[:, pl.ds(c0, C)], rbufs[b], rsem.at[b]).start()

        def rd_wait(b):
            pltpu.make_async_copy(
                tt_hbm.at[:, pl.ds(0, C)], rbufs[b], rsem.at[b]).wait()

        def wr_start(c0, b):
            pltpu.make_async_copy(
                sbufs[b].at[:, pl.ds(0, D)],
                out_hbm.at[pl.ds(c0, C)], wsem.at[b]).start()

        def wr_wait(b):
            pltpu.make_async_copy(
                sbufs[b].at[:, pl.ds(0, D)],
                out_hbm.at[pl.ds(0, C)], wsem.at[b]).wait()

        def transpose_chunk(b):
            src = rbufs[b]
            dst = sbufs[b]

            def cbody(c16, carry):
                cv = c16 * 16 + lane
                for f in range(D):
                    vals = src[f, pl.ds(c16 * 16, 16)]
                    plsc.store_scatter(dst, [cv, fsp[f]], vals)
                return carry

            lax.fori_loop(0, C // 16, cbody, 0)

        def run_chunk(kk, b):
            @pl.when(kk + 1 < nch)
            def _():
                rd_start(base + (kk + 1) * C, b ^ 1)
            rd_wait(b)

            @pl.when(kk >= 2)
            def _():
                wr_wait(b)
            transpose_chunk(b)
            wr_start(base + kk * C, b)

        rd_start(base, 0)

        def kbody(kb, carry):
            for u in range(2):
                run_chunk(kb * 2 + u, u)
            return carry

        lax.fori_loop(0, nch // 2, kbody, 0)
        wr_wait(0)
        wr_wait(1)

        @pl.when(wid == 0)
        def _():
            pltpu.sync_copy(tt_hbm.at[:, pl.ds(R - rem, rem)],
                            rb0.at[:, pl.ds(0, rem)])
            for q in range(rem // 16):
                cvq = q * 16 + lane
                for f in range(D):
                    vals = rb0[f, pl.ds(q * 16, 16)]
                    plsc.store_scatter(st0, [cvq, fsp[f]], vals)
            pltpu.sync_copy(st0.at[pl.ds(0, rem), pl.ds(0, D)],
                            out_hbm.at[pl.ds(R - rem, rem)])

    return k(table_t)


@functools.partial(jax.jit, static_argnums=(2, 3, 4))
def _sc_lookup(x_t, table, J, I, D):
    jpw = J // _NJ
    fb = D // 8
    mesh = plsc.VectorSubcoreMesh(core_axis_name="c", subcore_axis_name="s")

    @functools.partial(
        pl.kernel,
        mesh=mesh,
        out_type=jax.ShapeDtypeStruct((J, fb, I // 128, 8, 128), jnp.float32),
        scratch_types=[
            pltpu.VMEM((jpw, _G), jnp.int32),
            pltpu.VMEM((_G, D), jnp.float32),
            pltpu.VMEM((_G, D), jnp.float32),
            pltpu.VMEM((fb, _GB, 8, _P), jnp.float32),
            pltpu.VMEM((fb, _GB, 8, _P), jnp.float32),
            pltpu.SemaphoreType.DMA((2,)),
            pltpu.SemaphoreType.DMA((2,)),
        ],
        compiler_params=pltpu.CompilerParams(
            use_tc_tiling_on_sc=False, needs_layout_passes=False),
    )
    def k(xt_hbm, table_hbm, out_hbm, idx_v, rb0, rb1, tb0, tb1,
          gsem, wsem):
        wid = lax.axis_index("s") * _NC + lax.axis_index("c")
        ic0 = (wid % _NI) * _GB
        i0 = (wid % _NI) * _G
        jbase = (wid // _NI) * jpw
        rbufs = (rb0, rb1)
        tbufs = (tb0, tb1)
        lane = jnp.arange(16, dtype=jnp.int32)
        fbv = [(lane + 16 * h) // 8 for h in range(D // 16)]
        f8v = [(lane + 16 * h) % 8 for h in range(D // 16)]
        zero16 = jnp.zeros((16,), jnp.int32)

        pltpu.sync_copy(xt_hbm.at[pl.ds(jbase, jpw), pl.ds(i0, _G)], idx_v)

        def gather_start(jj, b):
            pltpu.make_async_copy(
                table_hbm.at[idx_v.at[jj]],
                rbufs[b], gsem.at[b]).start()

        def gather_wait(b):
            pltpu.make_async_copy(
                table_hbm.at[idx_v.at[0]],
                rbufs[b], gsem.at[b]).wait()

        def transpose(src, dst):
            for icl in range(_GB):
                icv = zero16 + icl

                def ibody(it, c, icl=icl, icv=icv):
                    base = it * 8
                    for u in range(8):
                        g = icl * 128 + base + u
                        i128v = zero16 + (base + u)
                        for h in range(D // 16):
                            vals = src[g, pl.ds(h * 16, 16)]
                            plsc.store_scatter(
                                dst, [fbv[h], icv, f8v[h], i128v], vals)
                    return c

                lax.fori_loop(0, 16, ibody, 0)

        def wb_start(jj, b):
            for f in range(fb):
                pltpu.make_async_copy(
                    tbufs[b].at[f, :, :, pl.ds(0, 128)],
                    out_hbm.at[jbase + jj, f, pl.ds(ic0, _GB), :, :],
                    wsem.at[b]).start()

        def wb_wait(b):
            for f in range(fb):
                pltpu.make_async_copy(
                    tbufs[b].at[f, :, :, pl.ds(0, 128)],
                    out_hbm.at[jbase, f, pl.ds(ic0, _GB), :, :],
                    wsem.at[b]).wait()

        gather_start(0, 0)

        def jblock(jb, carry):
            for u in range(2):
                jj = jb * 2 + u
                @pl.when(jj + 1 < jpw)
                def _():
                    gather_start(jj + 1, 1 - u)
                gather_wait(u)
                @pl.when(jj >= 2)
                def _():
                    wb_wait(u)
                transpose(rbufs[u], tbufs[u])
                wb_start(jj, u)
            return carry

        lax.fori_loop(0, jpw // 2, jblock, 0)
        wb_wait(0)
        wb_wait(1)

    return k(x_t, table)


def kernel(x, table):
    R, D = table.shape
    I, J = x.shape
    x_t = x.astype(jnp.int32).T
    table_lin = _sc_transpose(table.T, R, D)
    out5 = _sc_lookup(x_t, table_lin, J, I, D)
    return out5.transpose(2, 4, 0, 1, 3).reshape(I, J, D)


# speedup vs baseline: 4.3571x; 4.3571x over previous
import functools

import jax
import jax.numpy as jnp
from jax import lax
from jax.experimental import pallas as pl
from jax.experimental.pallas import tpu as pltpu
from jax.experimental.pallas import tpu_sc as plsc

_NC = 2
_NS = 16
_NW = _NC * _NS

_G = 512
_NI = 8
_NJ = 4
_GB = _G // 128
_P = 129


@functools.partial(jax.jit, static_argnums=(2, 3, 4))
def _sc_lookup(x_t, table, J, I, D):
    jpw = J // _NJ
    fb = D // 8
    mesh = plsc.VectorSubcoreMesh(core_axis_name="c", subcore_axis_name="s")

    @functools.partial(
        pl.kernel,
        mesh=mesh,
        out_type=jax.ShapeDtypeStruct((J, fb, I // 128, 8, 128), jnp.float32),
        scratch_types=[
            pltpu.VMEM((jpw, _G), jnp.int32),
            pltpu.VMEM((_G, D), jnp.float32),
            pltpu.VMEM((_G, D), jnp.float32),
            pltpu.VMEM((fb, _GB, 8, _P), jnp.float32),
            pltpu.VMEM((fb, _GB, 8, _P), jnp.float32),
            pltpu.SemaphoreType.DMA((2,)),
            pltpu.SemaphoreType.DMA((2,)),
        ],
        compiler_params=pltpu.CompilerParams(
            use_tc_tiling_on_sc=False, needs_layout_passes=False),
    )
    def k(xt_hbm, table_hbm, out_hbm, idx_v, rb0, rb1, tb0, tb1,
          gsem, wsem):
        wid = lax.axis_index("s") * _NC + lax.axis_index("c")
        ic0 = (wid % _NI) * _GB
        i0 = (wid % _NI) * _G
        jbase = (wid // _NI) * jpw
        rbufs = (rb0, rb1)
        tbufs = (tb0, tb1)
        lane = jnp.arange(16, dtype=jnp.int32)
        fbv = [(lane + 16 * h) // 8 for h in range(D // 16)]
        f8v = [(lane + 16 * h) % 8 for h in range(D // 16)]
        zero16 = jnp.zeros((16,), jnp.int32)

        pltpu.sync_copy(xt_hbm.at[pl.ds(jbase, jpw), pl.ds(i0, _G)], idx_v)

        def gather_start(jj, b):
            pltpu.make_async_copy(
                table_hbm.at[idx_v.at[jj]],
                rbufs[b], gsem.at[b]).start()

        def gather_wait(b):
            pltpu.make_async_copy(
                table_hbm.at[idx_v.at[0]],
                rbufs[b], gsem.at[b]).wait()

        def transpose(src, dst):
            for icl in range(_GB):
                icv = zero16 + icl

                def ibody(it, c, icl=icl, icv=icv):
                    base = it * 8
                    for u in range(8):
                        g = icl * 128 + base + u
                        i128v = zero16 + (base + u)
                        for h in range(D // 16):
                            vals = src[g, pl.ds(h * 16, 16)]
                            plsc.store_scatter(
                                dst, [fbv[h], icv, f8v[h], i128v], vals)
                    return c

                lax.fori_loop(0, 16, ibody, 0)

        def wb_start(jj, b):
            for f in range(fb):
                pltpu.make_async_copy(
                    tbufs[b].at[f, :, :, pl.ds(0, 128)],
                    out_hbm.at[jbase + jj, f, pl.ds(ic0, _GB), :, :],
                    wsem.at[b]).start()

        def wb_wait(b):
            for f in range(fb):
                pltpu.make_async_copy(
                    tbufs[b].at[f, :, :, pl.ds(0, 128)],
                    out_hbm.at[jbase, f, pl.ds(ic0, _GB), :, :],
                    wsem.at[b]).wait()

        gather_start(0, 0)

        def jblock(jb, carry):
            for u in range(2):
                jj = jb * 2 + u
                @pl.when(jj + 1 < jpw)
                def _():
                    gather_start(jj + 1, 1 - u)
                gather_wait(u)
                @pl.when(jj >= 2)
                def _():
                    wb_wait(u)
                transpose(rbufs[u], tbufs[u])
                wb_start(jj, u)
            return carry

        lax.fori_loop(0, jpw // 2, jblock, 0)
        wb_wait(0)
        wb_wait(1)

    return k(x_t, table)


def kernel(x, table):
    D = table.shape[1]
    I, J = x.shape
    x_t = x.astype(jnp.int32).T
    out5 = _sc_lookup(x_t, table, J, I, D)
    return out5.transpose(2, 4, 0, 1, 3).reshape(I, J, D)


# speedup vs baseline: 4.3897x; 1.0075x over previous
import functools

import jax
import jax.numpy as jnp
from jax import lax
from jax.experimental import pallas as pl
from jax.experimental.pallas import tpu as pltpu
from jax.experimental.pallas import tpu_sc as plsc

_NC = 2
_NS = 16
_NW = _NC * _NS

_G = 512
_NI = 8
_NJ = 4
_GB = _G // 128
_P = 129


@functools.partial(jax.jit, static_argnums=(2, 3, 4))
def _sc_lookup(x_t, table, J, I, D):
    jpw = J // _NJ
    fb = D // 8
    mesh = plsc.VectorSubcoreMesh(core_axis_name="c", subcore_axis_name="s")

    @functools.partial(
        pl.kernel,
        mesh=mesh,
        out_type=jax.ShapeDtypeStruct((J, fb, I // 128, 8, 128), jnp.float32),
        scratch_types=[
            pltpu.VMEM((jpw, _G), jnp.int32),
            pltpu.VMEM((_G, D), jnp.float32),
            pltpu.VMEM((_G, D), jnp.float32),
            pltpu.VMEM((fb, _GB, 10, _P), jnp.float32),
            pltpu.VMEM((fb, _GB, 10, _P), jnp.float32),
            pltpu.SemaphoreType.DMA((2,)),
            pltpu.SemaphoreType.DMA((2,)),
        ],
        compiler_params=pltpu.CompilerParams(
            use_tc_tiling_on_sc=False, needs_layout_passes=False),
    )
    def k(xt_hbm, table_hbm, out_hbm, idx_v, rb0, rb1, tb0, tb1,
          gsem, wsem):
        wid = lax.axis_index("s") * _NC + lax.axis_index("c")
        ic0 = (wid % _NI) * _GB
        i0 = (wid % _NI) * _G
        jbase = (wid // _NI) * jpw
        rbufs = (rb0, rb1)
        tbufs = (tb0, tb1)
        lane = jnp.arange(16, dtype=jnp.int32)
        fbv = [(lane + 16 * h) // 8 for h in range(D // 16)]
        f8v = [(lane + 16 * h) % 8 for h in range(D // 16)]
        zero16 = jnp.zeros((16,), jnp.int32)

        pltpu.sync_copy(xt_hbm.at[pl.ds(jbase, jpw), pl.ds(i0, _G)], idx_v)

        def gather_start(jj, b):
            pltpu.make_async_copy(
                table_hbm.at[idx_v.at[jj]],
                rbufs[b], gsem.at[b]).start()

        def gather_wait(b):
            pltpu.make_async_copy(
                table_hbm.at[idx_v.at[0]],
                rbufs[b], gsem.at[b]).wait()

        def transpose(src, dst):
            for icl in range(_GB):
                icv = zero16 + icl

                def ibody(it, c, icl=icl, icv=icv):
                    base = it * 8
                    for u in range(8):
                        g = icl * 128 + base + u
                        i128v = zero16 + (base + u)
                        for h in range(D // 16):
                            vals = src[g, pl.ds(h * 16, 16)]
                            plsc.store_scatter(
                                dst, [fbv[h], icv, f8v[h], i128v], vals)
                    return c

                lax.fori_loop(0, 16, ibody, 0)

        def wb_start(jj, b):
            for f in range(fb):
                pltpu.make_async_copy(
                    tbufs[b].at[f, :, pl.ds(0, 8), pl.ds(0, 128)],
                    out_hbm.at[jbase + jj, f, pl.ds(ic0, _GB), :, :],
                    wsem.at[b]).start()

        def wb_wait(b):
            for f in range(fb):
                pltpu.make_async_copy(
                    tbufs[b].at[f, :, pl.ds(0, 8), pl.ds(0, 128)],
                    out_hbm.at[jbase, f, pl.ds(ic0, _GB), :, :],
                    wsem.at[b]).wait()

        gather_start(0, 0)

        def jblock(jb, carry):
            for u in range(2):
                jj = jb * 2 + u
                @pl.when(jj + 1 < jpw)
                def _():
                    gather_start(jj + 1, 1 - u)
                gather_wait(u)
                @pl.when(jj >= 2)
                def _():
                    wb_wait(u)
                transpose(rbufs[u], tbufs[u])
                wb_start(jj, u)
            return carry

        lax.fori_loop(0, jpw // 2, jblock, 0)
        wb_wait(0)
        wb_wait(1)

    return k(x_t, table)


def kernel(x, table):
    D = table.shape[1]
    I, J = x.shape
    x_t = x.astype(jnp.int32).T
    out5 = _sc_lookup(x_t, table, J, I, D)
    return out5.transpose(2, 4, 0, 1, 3).reshape(I, J, D)
